# Initial kernel scaffold; baseline (speedup 1.0000x reference)
import jax
import jax.numpy as jnp
from jax.experimental import pallas as pl


def kernel(x, command):
    raise NotImplementedError("write your pallas kernel here")


# speedup vs baseline: 6.1205x; 6.1205x over previous
import functools

import jax
import jax.numpy as jnp
from jax.experimental import pallas as pl
from jax.experimental.pallas import tpu as pltpu

N_OUT = 4096
N_CMD = 4
BATCH = 4096

BB = 128
KCH = 512
NCH = KCH // N_CMD


def _tc_body(cmd_ref, x_ref, o_ref):
    c = cmd_ref[0]
    j = jax.lax.broadcasted_iota(jnp.int32, (KCH, NCH), 0)
    n = jax.lax.broadcasted_iota(jnp.int32, (KCH, NCH), 1)
    s = (j == N_CMD * n + c).astype(jnp.float32)
    for g in range(N_OUT * N_CMD // KCH):
        o_ref[:, g * NCH:(g + 1) * NCH] = jnp.dot(
            x_ref[:, g * KCH:(g + 1) * KCH], s,
            preferred_element_type=jnp.float32)


@functools.partial(jax.jit, static_argnames=("interpret",))
def kernel(x, command, interpret=False):
    grid_spec = pltpu.PrefetchScalarGridSpec(
        num_scalar_prefetch=1,
        grid=(BATCH // BB,),
        in_specs=[pl.BlockSpec((BB, N_OUT * N_CMD), lambda i, c: (i, 0))],
        out_specs=pl.BlockSpec((BB, N_OUT), lambda i, c: (i, 0)),
    )
    return pl.pallas_call(
        _tc_body,
        grid_spec=grid_spec,
        out_shape=jax.ShapeDtypeStruct((BATCH, N_OUT), jnp.float32),
        interpret=interpret,
    )(command, x)
